# baseline (speedup 1.0000x reference)
import functools

import jax
import jax.numpy as jnp
from jax import lax
from jax.experimental import pallas as pl
from jax.experimental.pallas import tpu as pltpu
from jax.experimental.pallas import tpu_sc as plsc

_NC = 2
_NS = 16
_NW = _NC * _NS

_B = 4 * 8192
_D = 1024
_BPW = _B // _NW
_C = 32
_NG = _BPW // _C
_NBUF = 3

_mesh = plsc.VectorSubcoreMesh(core_axis_name="c", subcore_axis_name="s")


@functools.partial(
    pl.kernel,
    mesh=_mesh,
    out_type=jax.ShapeDtypeStruct((_B, _D), jnp.float32),
    scratch_types=[
        pltpu.VMEM((_BPW,), jnp.int32),
        pltpu.VMEM((_NBUF, _C, _D), jnp.float32),
        pltpu.SemaphoreType.DMA,
        pltpu.SemaphoreType.DMA,
        pltpu.SemaphoreType.DMA,
    ],
)
def _gather_kernel(idx_hbm, table_hbm, out_hbm, idx_v, rows_v, gs0, gs1, wsem):
    wid = lax.axis_index("s") * _NC + lax.axis_index("c")
    base = wid * _BPW
    pltpu.sync_copy(idx_hbm.at[pl.ds(base, _BPW)], idx_v)

    def gather_desc(g, gsem):
        return pltpu.make_async_copy(
            table_hbm.at[idx_v.at[pl.ds(g * _C, _C)]],
            rows_v.at[lax.rem(g, _NBUF)],
            gsem,
        )

    def write_desc(g):
        return pltpu.make_async_copy(
            rows_v.at[lax.rem(g, _NBUF)],
            out_hbm.at[pl.ds(base + g * _C, _C)],
            wsem,
        )

    gather_desc(0, gs0).start()
    gather_desc(1, gs1).start()

    def body(g, gsem):
        gather_desc(g, gsem).wait()

        @pl.when(g >= 1)
        def _():
            write_desc(g - 1).wait()

        @pl.when(g < _NG - 2)
        def _():
            gather_desc(g + 2, gsem).start()

        write_desc(g).start()

    def step(g, _):
        r2 = lax.rem(g, 2)

        @pl.when(r2 == 0)
        def _():
            body(g, gs0)

        @pl.when(r2 == 1)
        def _():
            body(g, gs1)

        return ()

    lax.fori_loop(0, _NG, step, ())
    write_desc(_NG - 1).wait()


_RB = 256


def _tc_body(idx_ref, table_ref, out_ref):
    i = pl.program_id(0)
    for r in range(_RB):
        j = idx_ref[i * _RB + r]
        out_ref[pl.ds(r, 1)] = table_ref[pl.ds(j, 1)]


def _tc_gather(idx, table3, n_rows):
    grid_spec = pltpu.PrefetchScalarGridSpec(
        num_scalar_prefetch=1,
        grid=(n_rows // _RB,),
        in_specs=[
            pl.BlockSpec((8192, 8, 128), lambda i, idx_ref: (0, 0, 0)),
        ],
        out_specs=pl.BlockSpec((_RB, 8, 128), lambda i, idx_ref: (i, 0, 0)),
    )
    return pl.pallas_call(
        _tc_body,
        grid_spec=grid_spec,
        out_shape=jax.ShapeDtypeStruct((n_rows, 8, 128), jnp.float32),
    )(idx, table3)


def kernel(X, table):
    idx = X.reshape(-1).astype(jnp.int32)
    out = _gather_kernel(idx, table)
    return out.reshape(X.shape + (table.shape[1],))


# speedup vs baseline: 1.0058x; 1.0058x over previous
import functools

import jax
import jax.numpy as jnp
from jax import lax
from jax.experimental import pallas as pl
from jax.experimental.pallas import tpu as pltpu
from jax.experimental.pallas import tpu_sc as plsc

_NC = 2
_NS = 16
_NW = _NC * _NS

_B = 4 * 8192
_D = 1024
_BPW = _B // _NW
_C = 16
_NG = _BPW // _C
_NBUF = 7

_mesh = plsc.VectorSubcoreMesh(core_axis_name="c", subcore_axis_name="s")


@functools.partial(
    pl.kernel,
    mesh=_mesh,
    out_type=jax.ShapeDtypeStruct((_B, _D), jnp.float32),
    scratch_types=[
        pltpu.VMEM((_BPW,), jnp.int32),
        pltpu.VMEM((_NBUF, _C, _D), jnp.float32),
        pltpu.SemaphoreType.DMA,
        pltpu.SemaphoreType.DMA,
        pltpu.SemaphoreType.DMA,
        pltpu.SemaphoreType.DMA,
        pltpu.SemaphoreType.DMA,
        pltpu.SemaphoreType.DMA,
    ],
)
def _gather_kernel(
    idx_hbm, table_hbm, out_hbm, idx_v, rows_v, gs0, gs1, gs2, gs3, ws0, ws1
):
    wid = lax.axis_index("s") * _NC + lax.axis_index("c")
    base = wid * _BPW
    pltpu.sync_copy(idx_hbm.at[pl.ds(base, _BPW)], idx_v)

    def gather_desc(g, gsem):
        return pltpu.make_async_copy(
            table_hbm.at[idx_v.at[pl.ds(g * _C, _C)]],
            rows_v.at[lax.rem(g, _NBUF)],
            gsem,
        )

    def write_desc(g, wsem):
        return pltpu.make_async_copy(
            rows_v.at[lax.rem(g, _NBUF)],
            out_hbm.at[pl.ds(base + g * _C, _C)],
            wsem,
        )

    gss = (gs0, gs1, gs2, gs3)
    wss = (ws0, ws1)
    gather_desc(0, gs0).start()
    gather_desc(1, gs1).start()
    gather_desc(2, gs2).start()
    gather_desc(3, gs3).start()

    def body(g, gsem, wsem):
        gather_desc(g, gsem).wait()

        @pl.when(g >= 2)
        def _():
            write_desc(g - 2, wsem).wait()

        @pl.when(g < _NG - 4)
        def _():
            gather_desc(g + 4, gsem).start()

        write_desc(g, wsem).start()

    def step(g, _):
        r4 = lax.rem(g, 4)
        for a in range(4):

            @pl.when(r4 == a)
            def _(a=a):
                body(g, gss[a], wss[a % 2])

        return ()

    lax.fori_loop(0, _NG, step, ())
    write_desc(_NG - 2, ws0).wait()
    write_desc(_NG - 1, ws1).wait()


def kernel(X, table):
    idx = X.reshape(-1).astype(jnp.int32)
    out = _gather_kernel(idx, table)
    return out.reshape(X.shape + (table.shape[1],))
